# baseline (speedup 1.0000x reference)
import functools

import jax
import jax.numpy as jnp
from jax import lax
from jax.experimental import pallas as pl
from jax.experimental.pallas import tpu as pltpu
from jax.experimental.pallas import tpu_sc as plsc

NN = 10000
EE = 320000
DD = 128
GG = 64
CC = 10

NC = 2
NS = 16
NW = NC * NS
CH = 128
NCHUNK = 80
E_PAD = NW * NCHUNK * CH
N_PAD = 11776
RPT = N_PAD // NS
NBUF = 2
GAH = 1
SUP = 8
NSUP = NCHUNK // SUP

BLK = 400
NBLK = NN // BLK


def _segment_sum_sc(feat, idx4, zeros):
    mesh = plsc.VectorSubcoreMesh(core_axis_name="c", subcore_axis_name="s")

    @functools.partial(
        pl.kernel,
        out_type=jax.ShapeDtypeStruct((NC, N_PAD, DD), jnp.float32),
        mesh=mesh,
        scratch_types=[
            [pltpu.VMEM((2 * SUP, CH), jnp.int32) for _ in range(2)],
            [pltpu.VMEM((CH, DD), jnp.float32) for _ in range(NBUF)],
            pltpu.VMEM_SHARED((N_PAD, DD), jnp.float32),
            [pltpu.SemaphoreType.DMA for _ in range(2)],
            [pltpu.SemaphoreType.DMA for _ in range(NBUF)],
            [pltpu.SemaphoreType.DMA for _ in range(NBUF)],
        ],
    )
    def sc_kernel(x_hbm, idx_hbm, zero_hbm, out_hbm,
                  ibuf, rows_v, acc_sh, isems, gsems, ssems):
        cid = lax.axis_index("c")
        sid = lax.axis_index("s")
        wid = cid * NS + sid
        pltpu.sync_copy(zero_hbm, acc_sh.at[pl.ds(sid * RPT, RPT)])
        pltpu.sync_copy(idx_hbm.at[wid, 0], ibuf[0])
        pltpu.async_copy(idx_hbm.at[wid, 1], ibuf[1], isems[1])
        plsc.subcore_barrier()

        for b in range(GAH):
            pltpu.async_copy(x_hbm.at[ibuf[0].at[b]], rows_v[b], gsems[b])

        @pl.loop(0, NSUP, step=2)
        def _(s):
            for q in range(2):
                t = s + q
                for j in range(SUP):
                    b = j % NBUF
                    pltpu.make_async_copy(
                        x_hbm.at[ibuf[q].at[j]], rows_v[b], gsems[b]).wait()
                    pltpu.async_copy(rows_v[b], acc_sh.at[ibuf[q].at[SUP + j]],
                                     ssems[b], add=True)
                    b2 = (j + GAH) % NBUF
                    if j >= GAH:
                        pltpu.make_async_copy(
                            rows_v[b2], acc_sh.at[ibuf[q].at[SUP + j - GAH]],
                            ssems[b2]).wait()
                    else:
                        @pl.when(t > 0)
                        def _():
                            pltpu.make_async_copy(
                                rows_v[b2],
                                acc_sh.at[ibuf[1 - q].at[2 * SUP - GAH + j]],
                                ssems[b2]).wait()
                    if j < SUP - GAH:
                        pltpu.async_copy(x_hbm.at[ibuf[q].at[j + GAH]],
                                         rows_v[b2], gsems[b2])
                    else:
                        if j == SUP - GAH:
                            @pl.when(t + 1 < NSUP)
                            def _():
                                pltpu.make_async_copy(
                                    idx_hbm.at[wid, t + 1], ibuf[1 - q],
                                    isems[1 - q]).wait()

                        @pl.when(t + 1 < NSUP)
                        def _():
                            pltpu.async_copy(
                                x_hbm.at[ibuf[1 - q].at[j + GAH - SUP]],
                                rows_v[b2], gsems[b2])
                @pl.when(t + 2 < NSUP)
                def _():
                    pltpu.async_copy(idx_hbm.at[wid, t + 2], ibuf[q], isems[q])

        qlast = (NSUP - 1) % 2
        for k in range(GAH):
            c = NCHUNK - GAH + k
            pltpu.make_async_copy(
                rows_v[c % NBUF],
                acc_sh.at[ibuf[qlast].at[2 * SUP - GAH + k]],
                ssems[c % NBUF]).wait()

        plsc.subcore_barrier()
        pltpu.sync_copy(acc_sh.at[pl.ds(sid * RPT, RPT)],
                        out_hbm.at[cid, pl.ds(sid * RPT, RPT)])

    return sc_kernel(feat, idx4, zeros)


def _mlp1_tc(x, parts, W1, b1, W2, b2):
    def body(x_ref, p0_ref, p1_ref, w1_ref, b1_ref, w2_ref, b2_ref, o_ref):
        u = x_ref[...] + p0_ref[0] + p1_ref[0]
        t = jnp.dot(u, w1_ref[...], preferred_element_type=jnp.float32)
        t = jnp.maximum(t + b1_ref[...], 0.0)
        h = jnp.dot(t, w2_ref[...], preferred_element_type=jnp.float32)
        o_ref[...] = jnp.maximum(h + b2_ref[...], 0.0)

    return pl.pallas_call(
        body,
        grid=(NBLK,),
        in_specs=[
            pl.BlockSpec((BLK, DD), lambda i: (i, 0)),
            pl.BlockSpec((1, BLK, DD), lambda i: (0, i, 0)),
            pl.BlockSpec((1, BLK, DD), lambda i: (1, i, 0)),
            pl.BlockSpec((DD, DD), lambda i: (0, 0)),
            pl.BlockSpec((1, DD), lambda i: (0, 0)),
            pl.BlockSpec((DD, DD), lambda i: (0, 0)),
            pl.BlockSpec((1, DD), lambda i: (0, 0)),
        ],
        out_specs=pl.BlockSpec((BLK, DD), lambda i: (i, 0)),
        out_shape=jax.ShapeDtypeStruct((NN, DD), jnp.float32),
    )(x, parts, parts, W1, b1.reshape(1, DD), W2, b2.reshape(1, DD))


def _mlp2_pool_tc(h, parts, batch2d, W3, b3, W4, b4, Wl1, bl1, Wl2, bl2):
    def body(h_ref, p0_ref, p1_ref, b_ref, w3_ref, b3_ref, w4_ref, b4_ref,
             wl1_ref, bl1_ref, wl2_ref, bl2_ref, o_ref, acc_ref, cnt_ref):
        i = pl.program_id(0)

        @pl.when(i == 0)
        def _():
            acc_ref[...] = jnp.zeros_like(acc_ref)
            cnt_ref[...] = jnp.zeros_like(cnt_ref)

        u = h_ref[...] + p0_ref[0] + p1_ref[0]
        t = jnp.dot(u, w3_ref[...], preferred_element_type=jnp.float32)
        t = jnp.maximum(t + b3_ref[...], 0.0)
        h2 = jnp.dot(t, w4_ref[...], preferred_element_type=jnp.float32)
        h2 = h2 + b4_ref[...]

        gids = lax.broadcasted_iota(jnp.int32, (1, GG), 1)
        onehot = (b_ref[...] == gids).astype(jnp.float32)
        acc_ref[...] += lax.dot_general(
            onehot, h2, (((0,), (0,)), ((), ())),
            preferred_element_type=jnp.float32)
        cnt_ref[...] += lax.dot_general(
            onehot, jnp.ones((BLK, DD), jnp.float32), (((0,), (0,)), ((), ())),
            preferred_element_type=jnp.float32)

        @pl.when(i == NBLK - 1)
        def _():
            pooled = acc_ref[...] / jnp.maximum(cnt_ref[...], 1.0)
            r = jnp.dot(pooled, wl1_ref[...],
                        preferred_element_type=jnp.float32) + bl1_ref[...]
            o_ref[...] = jnp.dot(r, wl2_ref[...],
                                 preferred_element_type=jnp.float32) + bl2_ref[...]

    return pl.pallas_call(
        body,
        grid=(NBLK,),
        in_specs=[
            pl.BlockSpec((BLK, DD), lambda i: (i, 0)),
            pl.BlockSpec((1, BLK, DD), lambda i: (0, i, 0)),
            pl.BlockSpec((1, BLK, DD), lambda i: (1, i, 0)),
            pl.BlockSpec((BLK, 1), lambda i: (i, 0)),
            pl.BlockSpec((DD, DD), lambda i: (0, 0)),
            pl.BlockSpec((1, DD), lambda i: (0, 0)),
            pl.BlockSpec((DD, DD), lambda i: (0, 0)),
            pl.BlockSpec((1, DD), lambda i: (0, 0)),
            pl.BlockSpec((DD, DD // 2), lambda i: (0, 0)),
            pl.BlockSpec((1, DD // 2), lambda i: (0, 0)),
            pl.BlockSpec((DD // 2, CC), lambda i: (0, 0)),
            pl.BlockSpec((1, CC), lambda i: (0, 0)),
        ],
        out_specs=pl.BlockSpec((GG, CC), lambda i: (0, 0)),
        out_shape=jax.ShapeDtypeStruct((GG, CC), jnp.float32),
        scratch_shapes=[
            pltpu.VMEM((GG, DD), jnp.float32),
            pltpu.VMEM((GG, DD), jnp.float32),
        ],
    )(h, parts, parts, batch2d, W3, b3.reshape(1, DD), W4, b4.reshape(1, DD),
      Wl1, bl1.reshape(1, DD // 2), Wl2, bl2.reshape(1, CC))


def kernel(x, edge_index, batch, W1, b1, W2, b2, W3, b3, W4, b4,
           Wl1, bl1, Wl2, bl2):
    ppt = (E_PAD - EE) // NW
    pad_src = (jnp.arange(NW, dtype=jnp.int32)[:, None] * ppt
               + jnp.arange(ppt, dtype=jnp.int32)[None, :]) % NN
    src_p = jnp.concatenate(
        [edge_index[0].reshape(NW, EE // NW), pad_src], axis=1)
    rows_per_tile = (N_PAD - NN) // NS
    pad_dst = (NN
               + (jnp.arange(NW, dtype=jnp.int32) % NS)[:, None] * rows_per_tile
               + (jnp.arange(ppt, dtype=jnp.int32) % rows_per_tile)[None, :])
    dst_p = jnp.concatenate(
        [edge_index[1].reshape(NW, EE // NW), pad_dst], axis=1)
    src4 = src_p.reshape(NW, NSUP, SUP, CH)
    dst4 = dst_p.reshape(NW, NSUP, SUP, CH)
    idx4 = jnp.concatenate([src4, dst4], axis=2)
    zeros = jnp.zeros((RPT, DD), jnp.float32)
    batch2d = batch.reshape(NN, 1)

    parts1 = _segment_sum_sc(x, idx4, zeros)
    h = _mlp1_tc(x, parts1, W1, b1, W2, b2)
    parts2 = _segment_sum_sc(h, idx4, zeros)
    return _mlp2_pool_tc(h, parts2, batch2d, W3, b3, W4, b4, Wl1, bl1, Wl2, bl2)


# speedup vs baseline: 1.2236x; 1.2236x over previous
import functools

import jax
import jax.numpy as jnp
from jax import lax
from jax.experimental import pallas as pl
from jax.experimental.pallas import tpu as pltpu
from jax.experimental.pallas import tpu_sc as plsc

NN = 10000
EE = 320000
DD = 128
GG = 64
CC = 10

NC = 2
NS = 16
NW = NC * NS
CH = 112
NCHUNK = 96
E_PAD = NW * NCHUNK * CH
N_PAD = 10112
RPT = N_PAD // NS
NBUF = 3
SUP = 6
NSUP = NCHUNK // SUP

BLK = 400
NBLK = NN // BLK


def _segment_sum_sc(feat, idx4, zeros):
    mesh = plsc.VectorSubcoreMesh(core_axis_name="c", subcore_axis_name="s")

    @functools.partial(
        pl.kernel,
        out_type=jax.ShapeDtypeStruct((NC, N_PAD, DD), jnp.float32),
        mesh=mesh,
        scratch_types=[
            [pltpu.VMEM((2 * SUP, CH), jnp.int32) for _ in range(2)],
            [pltpu.VMEM((CH, DD), jnp.float32) for _ in range(NBUF)],
            pltpu.VMEM_SHARED((N_PAD, DD), jnp.float32),
            [pltpu.SemaphoreType.DMA for _ in range(2)],
            [pltpu.SemaphoreType.DMA for _ in range(NBUF)],
        ],
    )
    def sc_kernel(x_hbm, idx_hbm, zero_hbm, out_hbm,
                  ibuf, rows_v, acc_sh, isems, rsems):
        cid = lax.axis_index("c")
        sid = lax.axis_index("s")
        wid = cid * NS + sid
        pltpu.sync_copy(zero_hbm, acc_sh.at[pl.ds(sid * RPT, RPT)])
        pltpu.sync_copy(idx_hbm.at[wid, 0], ibuf[0])
        pltpu.async_copy(idx_hbm.at[wid, 1], ibuf[1], isems[1])
        plsc.subcore_barrier()

        for b in range(NBUF):
            pltpu.async_copy(x_hbm.at[ibuf[0].at[b]], rows_v[b], rsems[b])

        @pl.loop(0, NSUP, step=2)
        def _(s):
            for q in range(2):
                t = s + q
                for j in range(SUP):
                    b = j % NBUF
                    pltpu.make_async_copy(
                        x_hbm.at[ibuf[q].at[j]], rows_v[b], rsems[b]).wait()
                    pltpu.sync_copy(rows_v[b], acc_sh.at[ibuf[q].at[SUP + j]],
                                    add=True)
                    if j < SUP - NBUF:
                        pltpu.async_copy(x_hbm.at[ibuf[q].at[j + NBUF]],
                                         rows_v[b], rsems[b])
                    else:
                        if j == SUP - NBUF:
                            @pl.when(t + 1 < NSUP)
                            def _():
                                pltpu.make_async_copy(
                                    idx_hbm.at[wid, t + 1], ibuf[1 - q],
                                    isems[1 - q]).wait()

                        @pl.when(t + 1 < NSUP)
                        def _():
                            pltpu.async_copy(
                                x_hbm.at[ibuf[1 - q].at[j + NBUF - SUP]],
                                rows_v[b], rsems[b])
                @pl.when(t + 2 < NSUP)
                def _():
                    pltpu.async_copy(idx_hbm.at[wid, t + 2], ibuf[q], isems[q])

        plsc.subcore_barrier()
        pltpu.sync_copy(acc_sh.at[pl.ds(sid * RPT, RPT)],
                        out_hbm.at[cid, pl.ds(sid * RPT, RPT)])

    return sc_kernel(feat, idx4, zeros)


def _mlp1_tc(x, parts, W1, b1, W2, b2):
    def body(x_ref, p0_ref, p1_ref, w1_ref, b1_ref, w2_ref, b2_ref, o_ref):
        u = x_ref[...] + p0_ref[0] + p1_ref[0]
        t = jnp.dot(u, w1_ref[...], preferred_element_type=jnp.float32)
        t = jnp.maximum(t + b1_ref[...], 0.0)
        h = jnp.dot(t, w2_ref[...], preferred_element_type=jnp.float32)
        o_ref[...] = jnp.maximum(h + b2_ref[...], 0.0)

    return pl.pallas_call(
        body,
        grid=(NBLK,),
        in_specs=[
            pl.BlockSpec((BLK, DD), lambda i: (i, 0)),
            pl.BlockSpec((1, BLK, DD), lambda i: (0, i, 0)),
            pl.BlockSpec((1, BLK, DD), lambda i: (1, i, 0)),
            pl.BlockSpec((DD, DD), lambda i: (0, 0)),
            pl.BlockSpec((1, DD), lambda i: (0, 0)),
            pl.BlockSpec((DD, DD), lambda i: (0, 0)),
            pl.BlockSpec((1, DD), lambda i: (0, 0)),
        ],
        out_specs=pl.BlockSpec((BLK, DD), lambda i: (i, 0)),
        out_shape=jax.ShapeDtypeStruct((NN, DD), jnp.float32),
    )(x, parts, parts, W1, b1.reshape(1, DD), W2, b2.reshape(1, DD))


def _mlp2_pool_tc(h, parts, batch2d, W3, b3, W4, b4, Wl1, bl1, Wl2, bl2):
    def body(h_ref, p0_ref, p1_ref, b_ref, w3_ref, b3_ref, w4_ref, b4_ref,
             wl1_ref, bl1_ref, wl2_ref, bl2_ref, o_ref, acc_ref, cnt_ref):
        i = pl.program_id(0)

        @pl.when(i == 0)
        def _():
            acc_ref[...] = jnp.zeros_like(acc_ref)
            cnt_ref[...] = jnp.zeros_like(cnt_ref)

        u = h_ref[...] + p0_ref[0] + p1_ref[0]
        t = jnp.dot(u, w3_ref[...], preferred_element_type=jnp.float32)
        t = jnp.maximum(t + b3_ref[...], 0.0)
        h2 = jnp.dot(t, w4_ref[...], preferred_element_type=jnp.float32)
        h2 = h2 + b4_ref[...]

        gids = lax.broadcasted_iota(jnp.int32, (1, GG), 1)
        onehot = (b_ref[...] == gids).astype(jnp.float32)
        acc_ref[...] += lax.dot_general(
            onehot, h2, (((0,), (0,)), ((), ())),
            preferred_element_type=jnp.float32)
        cnt_ref[...] += lax.dot_general(
            onehot, jnp.ones((BLK, DD), jnp.float32), (((0,), (0,)), ((), ())),
            preferred_element_type=jnp.float32)

        @pl.when(i == NBLK - 1)
        def _():
            pooled = acc_ref[...] / jnp.maximum(cnt_ref[...], 1.0)
            r = jnp.dot(pooled, wl1_ref[...],
                        preferred_element_type=jnp.float32) + bl1_ref[...]
            o_ref[...] = jnp.dot(r, wl2_ref[...],
                                 preferred_element_type=jnp.float32) + bl2_ref[...]

    return pl.pallas_call(
        body,
        grid=(NBLK,),
        in_specs=[
            pl.BlockSpec((BLK, DD), lambda i: (i, 0)),
            pl.BlockSpec((1, BLK, DD), lambda i: (0, i, 0)),
            pl.BlockSpec((1, BLK, DD), lambda i: (1, i, 0)),
            pl.BlockSpec((BLK, 1), lambda i: (i, 0)),
            pl.BlockSpec((DD, DD), lambda i: (0, 0)),
            pl.BlockSpec((1, DD), lambda i: (0, 0)),
            pl.BlockSpec((DD, DD), lambda i: (0, 0)),
            pl.BlockSpec((1, DD), lambda i: (0, 0)),
            pl.BlockSpec((DD, DD // 2), lambda i: (0, 0)),
            pl.BlockSpec((1, DD // 2), lambda i: (0, 0)),
            pl.BlockSpec((DD // 2, CC), lambda i: (0, 0)),
            pl.BlockSpec((1, CC), lambda i: (0, 0)),
        ],
        out_specs=pl.BlockSpec((GG, CC), lambda i: (0, 0)),
        out_shape=jax.ShapeDtypeStruct((GG, CC), jnp.float32),
        scratch_shapes=[
            pltpu.VMEM((GG, DD), jnp.float32),
            pltpu.VMEM((GG, DD), jnp.float32),
        ],
    )(h, parts, parts, batch2d, W3, b3.reshape(1, DD), W4, b4.reshape(1, DD),
      Wl1, bl1.reshape(1, DD // 2), Wl2, bl2.reshape(1, CC))


def kernel(x, edge_index, batch, W1, b1, W2, b2, W3, b3, W4, b4,
           Wl1, bl1, Wl2, bl2):
    ppt = (E_PAD - EE) // NW
    pad_src = (jnp.arange(NW, dtype=jnp.int32)[:, None] * ppt
               + jnp.arange(ppt, dtype=jnp.int32)[None, :]) % NN
    src_p = jnp.concatenate(
        [edge_index[0].reshape(NW, EE // NW), pad_src], axis=1)
    rows_per_tile = (N_PAD - NN) // NS
    pad_dst = (NN
               + (jnp.arange(NW, dtype=jnp.int32) % NS)[:, None] * rows_per_tile
               + (jnp.arange(ppt, dtype=jnp.int32) % rows_per_tile)[None, :])
    dst_p = jnp.concatenate(
        [edge_index[1].reshape(NW, EE // NW), pad_dst], axis=1)
    src4 = src_p.reshape(NW, NSUP, SUP, CH)
    dst4 = dst_p.reshape(NW, NSUP, SUP, CH)
    idx4 = jnp.concatenate([src4, dst4], axis=2)
    zeros = jnp.zeros((RPT, DD), jnp.float32)
    batch2d = batch.reshape(NN, 1)

    parts1 = _segment_sum_sc(x, idx4, zeros)
    h = _mlp1_tc(x, parts1, W1, b1, W2, b2)
    parts2 = _segment_sum_sc(h, idx4, zeros)
    return _mlp2_pool_tc(h, parts2, batch2d, W3, b3, W4, b4, Wl1, bl1, Wl2, bl2)
